# baseline (speedup 1.0000x reference)
import jax
import jax.numpy as jnp
from jax import lax
from jax.experimental import pallas as pl
from jax.experimental.pallas import tpu as pltpu

B = 4096
L = 2048
BR = 256
GRID = B // BR


def _ave_body(yt_ref, out_ref):
    y0 = yt_ref[0:1, :]
    y1 = yt_ref[1:2, :]
    pen = (jnp.maximum(1.5 - y0, 0.0) + jnp.maximum(y0 - 4.0, 0.0)
           + jnp.maximum(1.0 - y1, 0.0) + jnp.maximum(y1 - 5.0, 0.0))
    out_ref[0, 0] = jnp.sum(pen)


def _sinr_body(y_ref, x_ref, p_ref, out_ref, acc_ref):
    i = pl.program_id(0)

    @pl.when(i == 0)
    def _init():
        acc_ref[0] = 0.0

    x = x_ref[...]
    p = p_ref[...]
    ys = y_ref[pl.ds(i * BR, BR), :]
    y0c = ys[:, 0:1]
    y1c = ys[:, 1:2]
    xj = jnp.abs(x)
    flag_t = xj <= y1c
    flag_at = (xj <= y0c * y1c) & (xj > y1c)
    sig = jnp.where(flag_t, x, 0.0) + flag_at.astype(jnp.float32) * y1c
    n = sig - p
    pn_s = jnp.sum(n * n, axis=1)
    ps_s = jnp.sum(p * p, axis=1)
    acc_ref[0] += jnp.sum(pn_s / ps_s)

    @pl.when(i == GRID - 1)
    def _fin():
        out_ref[0, 0] = acc_ref[0] / B


def _sinr_heavy(ops):
    y_, x_, p_ = ops
    x2 = x_.reshape(B, L)
    out = pl.pallas_call(
        _sinr_body,
        grid=(GRID,),
        in_specs=[
            pl.BlockSpec(memory_space=pltpu.VMEM),
            pl.BlockSpec((BR, L), lambda i: (i, 0)),
            pl.BlockSpec((BR, L), lambda i: (i, 0)),
        ],
        out_specs=pl.BlockSpec(memory_space=pltpu.SMEM),
        out_shape=jax.ShapeDtypeStruct((1, 1), jnp.float32),
        scratch_shapes=[pltpu.SMEM((1,), jnp.float32)],
    )(y_, x2, p_)
    return out[0, 0]


def kernel(y, x, p):
    ave = pl.pallas_call(
        _ave_body,
        out_specs=pl.BlockSpec(memory_space=pltpu.SMEM),
        out_shape=jax.ShapeDtypeStruct((1, 1), jnp.float32),
    )(y.T)[0, 0]
    return lax.cond(ave != 0.0, lambda ops: ave, _sinr_heavy, (y, x, p))
